# Initial kernel scaffold; baseline (speedup 1.0000x reference)
import jax
import jax.numpy as jnp
from jax.experimental import pallas as pl


def kernel(x, edge_weight, W1, b1, W2, b2, edge_index):
    raise NotImplementedError("write your pallas kernel here")


# speedup vs baseline: 13.9398x; 13.9398x over previous
import functools

import jax
import jax.numpy as jnp
from jax import lax
from jax.experimental import pallas as pl
from jax.experimental.pallas import tpu as pltpu
from jax.experimental.pallas import tpu_sc as plsc

NC = 2
NS = 16
NW = NC * NS
LANES = 16
C = 128


def _sc_mesh():
    return plsc.VectorSubcoreMesh(
        core_axis_name="c", subcore_axis_name="s", num_cores=NC,
        num_subcores=NS)


def _deg_call(dst3, ew3, n_pad, nchunk):
    stripe = n_pad // NS

    def body(dst_hbm, ew_hbm, out_hbm, deg_sh, deg_loc, dst_v, ew_v,
             red_v, zbuf):
        cid = lax.axis_index("c")
        sid = lax.axis_index("s")
        wid = sid * NC + cid

        def zloop(i, carry):
            deg_loc[pl.ds(i * LANES, LANES)] = jnp.zeros((LANES,),
                                                         jnp.float32)
            return carry
        lax.fori_loop(0, n_pad // LANES, zloop, 0)
        pltpu.sync_copy(dst_hbm.at[wid], dst_v)
        pltpu.sync_copy(ew_hbm.at[wid], ew_v)

        def chunk(i, carry):
            for b in range(C // LANES):
                sl = pl.ds(b * LANES, LANES)
                plsc.addupdate_scatter(deg_loc, [dst_v[i, sl]], ew_v[i, sl])
            return carry
        lax.fori_loop(0, nchunk, chunk, 0)
        pltpu.sync_copy(deg_loc, deg_sh.at[sid])
        plsc.subcore_barrier()
        pltpu.sync_copy(deg_sh.at[:, pl.ds(sid * stripe, stripe)], red_v)

        def red(j, carry):
            sl = pl.ds(j * LANES, LANES)
            acc = red_v[0, sl]
            for t in range(1, NS):
                acc = acc + red_v[t, sl]
            zbuf[sl] = acc
            return carry
        lax.fori_loop(0, stripe // LANES, red, 0)
        pltpu.sync_copy(zbuf,
                        out_hbm.at[pl.ds(cid * n_pad + sid * stripe, stripe)])

    return pl.kernel(
        body,
        out_type=jax.ShapeDtypeStruct((NC * n_pad,), jnp.float32),
        mesh=_sc_mesh(),
        compiler_params=pltpu.CompilerParams(needs_layout_passes=False),
        scratch_types=[
            pltpu.VMEM_SHARED((NS, n_pad), jnp.float32),
            pltpu.VMEM((n_pad,), jnp.float32),
            pltpu.VMEM((nchunk, C), jnp.int32),
            pltpu.VMEM((nchunk, C), jnp.float32),
            pltpu.VMEM((NS, stripe), jnp.float32),
            pltpu.VMEM((stripe,), jnp.float32),
        ],
    )(dst3, ew3)


def _dinv_call(degp, n_pad):
    deg3 = degp.reshape(NC, n_pad // 128, 128)

    def body(d_ref, dinv_ref, dinv2_ref):
        deg = d_ref[0] + d_ref[1] + 1.0
        di = lax.rsqrt(deg)
        dinv_ref[...] = di
        dinv2_ref[...] = di * di

    shp = jax.ShapeDtypeStruct((n_pad // 128, 128), jnp.float32)
    dinv3, dinv23 = pl.pallas_call(body, out_shape=(shp, shp))(deg3)
    return dinv3.reshape(n_pad), dinv23.reshape(n_pad)


def _norm_call(src3, dst3, ew3, dinv, n_pad, nchunk):
    def body(src_hbm, dst_hbm, ew_hbm, dinv_hbm, norm_hbm,
             dinv_v, src_v, dst_v, ew_v, norm_v):
        cid = lax.axis_index("c")
        sid = lax.axis_index("s")
        wid = sid * NC + cid
        pltpu.sync_copy(dinv_hbm, dinv_v)
        pltpu.sync_copy(src_hbm.at[wid], src_v)
        pltpu.sync_copy(dst_hbm.at[wid], dst_v)
        pltpu.sync_copy(ew_hbm.at[wid], ew_v)

        def chunk(i, carry):
            for l in range(C // LANES):
                sl = pl.ds(l * LANES, LANES)
                sv = src_v[i, sl]
                dv = dst_v[i, sl]
                ev = ew_v[i, sl]
                nv = (plsc.load_gather(dinv_v, [sv]) * ev
                      * plsc.load_gather(dinv_v, [dv]))
                norm_v[i, sl] = nv
            return carry
        lax.fori_loop(0, nchunk, chunk, 0)
        pltpu.sync_copy(norm_v, norm_hbm.at[wid])

    return pl.kernel(
        body,
        out_type=jax.ShapeDtypeStruct((NW, nchunk, C), jnp.float32),
        mesh=_sc_mesh(),
        compiler_params=pltpu.CompilerParams(needs_layout_passes=False),
        scratch_types=[
            pltpu.VMEM((n_pad,), jnp.float32),
            pltpu.VMEM((nchunk, C), jnp.int32),
            pltpu.VMEM((nchunk, C), jnp.int32),
            pltpu.VMEM((nchunk, C), jnp.float32),
            pltpu.VMEM((nchunk, C), jnp.float32),
        ],
    )(src3, dst3, ew3, dinv)


def _agg_call(xin, src3, dst3, norm3, n_pad, d, nchunk):
    rpt = n_pad // NS

    stripe_chunks = []
    off = 0
    while off < rpt:
        sz = min(C, rpt - off)
        stripe_chunks.append((off, sz))
        off += sz

    def body(x_hbm, src_hbm, dst_hbm, norm_hbm, out_hbm,
             agg_sh, src_v, dst_v, norm_v, rows, sem):
        cid = lax.axis_index("c")
        sid = lax.axis_index("s")
        wid = sid * NC + cid
        def zloop(r, carry):
            for l in range(d // LANES):
                rows[r, pl.ds(l * LANES, LANES)] = jnp.zeros((LANES,),
                                                             jnp.float32)
            return carry
        lax.fori_loop(0, C, zloop, 0)
        for off, sz in stripe_chunks:
            pltpu.sync_copy(rows.at[pl.ds(0, sz)],
                            agg_sh.at[pl.ds(sid * rpt + off, sz)])
        pltpu.sync_copy(src_hbm.at[wid], src_v)
        pltpu.sync_copy(dst_hbm.at[wid], dst_v)
        pltpu.sync_copy(norm_hbm.at[wid], norm_v)
        plsc.subcore_barrier()

        def chunk(i, carry):
            pltpu.async_copy(x_hbm.at[src_v.at[i]], rows, sem).wait()

            def rloop(b, c2):
                nv = norm_v[i, pl.ds(b * LANES, LANES)]
                for j in range(LANES):
                    r = b * LANES + j
                    nrm = nv[j]
                    for l in range(d // LANES):
                        sl = pl.ds(l * LANES, LANES)
                        rows[r, sl] = rows[r, sl] * nrm
                return c2
            lax.fori_loop(0, C // LANES, rloop, 0)
            pltpu.sync_copy(rows, agg_sh.at[dst_v.at[i]], add=True)
            return carry
        lax.fori_loop(0, nchunk, chunk, 0)
        plsc.subcore_barrier()
        for off, sz in stripe_chunks:
            pltpu.sync_copy(agg_sh.at[pl.ds(sid * rpt + off, sz)],
                            rows.at[pl.ds(0, sz)])
            pltpu.sync_copy(rows.at[pl.ds(0, sz)],
                            out_hbm.at[cid, pl.ds(sid * rpt + off, sz)])

    return pl.kernel(
        body,
        out_type=jax.ShapeDtypeStruct((NC, n_pad, d), jnp.float32),
        mesh=_sc_mesh(),
        compiler_params=pltpu.CompilerParams(needs_layout_passes=False),
        scratch_types=[
            pltpu.VMEM_SHARED((n_pad, d), jnp.float32),
            pltpu.VMEM((nchunk, C), jnp.int32),
            pltpu.VMEM((nchunk, C), jnp.int32),
            pltpu.VMEM((nchunk, C), jnp.float32),
            pltpu.VMEM((C, d), jnp.float32),
            pltpu.SemaphoreType.DMA,
        ],
    )(xin, src3, dst3, norm3)


def _mm_call(p, xin, dinv2_col, w, brow, relu, n, d):
    rb = 1000
    grid = (n // rb,)

    def body(p_ref, x_ref, d2_ref, w_ref, b_ref, o_ref):
        acc = p_ref[0] + p_ref[1] + x_ref[...] * d2_ref[...]
        y = jnp.dot(acc, w_ref[...], preferred_element_type=jnp.float32)
        y = y + b_ref[...]
        if relu:
            y = jnp.maximum(y, 0.0)
        o_ref[...] = y

    return pl.pallas_call(
        body,
        grid=grid,
        in_specs=[
            pl.BlockSpec((NC, rb, d), lambda i: (0, i, 0)),
            pl.BlockSpec((rb, d), lambda i: (i, 0)),
            pl.BlockSpec((rb, 1), lambda i: (i, 0)),
            pl.BlockSpec((d, d), lambda i: (0, 0)),
            pl.BlockSpec((1, d), lambda i: (0, 0)),
        ],
        out_specs=pl.BlockSpec((rb, d), lambda i: (i, 0)),
        out_shape=jax.ShapeDtypeStruct((n, d), jnp.float32),
    )(p, xin, dinv2_col, w, brow)


def kernel(x, edge_weight, W1, b1, W2, b2, edge_index):
    n, d = x.shape
    e = edge_index.shape[1]
    assert n % NS == 0 and d % LANES == 0
    ept = ((e + NW * C - 1) // (NW * C)) * C
    nchunk = ept // C
    e_pad = ept * NW
    n_pad = ((n + NS * LANES - 1) // (NS * LANES)) * (NS * LANES)

    src = edge_index[0]
    dst = edge_index[1]
    npad_e = e_pad - e
    pad_ids = (jnp.arange(npad_e, dtype=jnp.int32) % n)
    src3 = jnp.concatenate([src, pad_ids]).reshape(NW, nchunk, C)
    dst3 = jnp.concatenate([dst, pad_ids]).reshape(NW, nchunk, C)
    ew3 = jnp.concatenate(
        [edge_weight, jnp.zeros((npad_e,), jnp.float32)]).reshape(NW, nchunk, C)

    degp = _deg_call(dst3, ew3, n_pad, nchunk).reshape(NC, n_pad)
    dinv, dinv2 = _dinv_call(degp, n_pad)
    norm3 = _norm_call(src3, dst3, ew3, dinv, n_pad, nchunk)

    dinv2_col = dinv2[:n].reshape(n, 1)
    b1row = b1.reshape(1, d)
    b2row = b2.reshape(1, d)

    p1 = _agg_call(x, src3, dst3, norm3, n_pad, d, nchunk)
    h = _mm_call(p1, x, dinv2_col, W1, b1row, True, n, d)
    p2 = _agg_call(h, src3, dst3, norm3, n_pad, d, nchunk)
    out = _mm_call(p2, h, dinv2_col, W2, b2row, False, n, d)
    return out


# speedup vs baseline: 21.0622x; 1.5109x over previous
import functools

import jax
import jax.numpy as jnp
from jax import lax
from jax.experimental import pallas as pl
from jax.experimental.pallas import tpu as pltpu
from jax.experimental.pallas import tpu_sc as plsc

NC = 2
NS = 16
NW = NC * NS
LANES = 16
C = 128


def _sc_mesh():
    return plsc.VectorSubcoreMesh(
        core_axis_name="c", subcore_axis_name="s", num_cores=NC,
        num_subcores=NS)


def _deg_call(dst3, ew3, n_pad, nchunk):
    stripe = n_pad // NS

    def body(dst_hbm, ew_hbm, out_hbm, deg_sh, deg_loc, dst_v, ew_v,
             red_v, zbuf):
        cid = lax.axis_index("c")
        sid = lax.axis_index("s")
        wid = sid * NC + cid

        def zloop(i, carry):
            deg_loc[pl.ds(i * LANES, LANES)] = jnp.zeros((LANES,),
                                                         jnp.float32)
            return carry
        lax.fori_loop(0, n_pad // LANES, zloop, 0)
        pltpu.sync_copy(dst_hbm.at[wid], dst_v)
        pltpu.sync_copy(ew_hbm.at[wid], ew_v)

        def chunk(i, carry):
            for b in range(C // LANES):
                sl = pl.ds(b * LANES, LANES)
                plsc.addupdate_scatter(deg_loc, [dst_v[i, sl]], ew_v[i, sl])
            return carry
        lax.fori_loop(0, nchunk, chunk, 0)
        pltpu.sync_copy(deg_loc, deg_sh.at[sid])
        plsc.subcore_barrier()
        pltpu.sync_copy(deg_sh.at[:, pl.ds(sid * stripe, stripe)], red_v)

        def red(j, carry):
            sl = pl.ds(j * LANES, LANES)
            acc = red_v[0, sl]
            for t in range(1, NS):
                acc = acc + red_v[t, sl]
            zbuf[sl] = acc
            return carry
        lax.fori_loop(0, stripe // LANES, red, 0)
        pltpu.sync_copy(zbuf,
                        out_hbm.at[pl.ds(cid * n_pad + sid * stripe, stripe)])

    return pl.kernel(
        body,
        out_type=jax.ShapeDtypeStruct((NC * n_pad,), jnp.float32),
        mesh=_sc_mesh(),
        compiler_params=pltpu.CompilerParams(needs_layout_passes=False),
        scratch_types=[
            pltpu.VMEM_SHARED((NS, n_pad), jnp.float32),
            pltpu.VMEM((n_pad,), jnp.float32),
            pltpu.VMEM((nchunk, C), jnp.int32),
            pltpu.VMEM((nchunk, C), jnp.float32),
            pltpu.VMEM((NS, stripe), jnp.float32),
            pltpu.VMEM((stripe,), jnp.float32),
        ],
    )(dst3, ew3)


def _dinv_call(degp, n_pad):
    deg3 = degp.reshape(NC, n_pad // 128, 128)

    def body(d_ref, dinv_ref, dinv2_ref):
        deg = d_ref[0] + d_ref[1] + 1.0
        di = lax.rsqrt(deg)
        dinv_ref[...] = di
        dinv2_ref[...] = di * di

    shp = jax.ShapeDtypeStruct((n_pad // 128, 128), jnp.float32)
    dinv3, dinv23 = pl.pallas_call(body, out_shape=(shp, shp))(deg3)
    return dinv3.reshape(n_pad), dinv23.reshape(n_pad)


def _prescale_call(x, dinv_col, n, d):
    rb = 1000

    def body(x_ref, dc_ref, o_ref):
        o_ref[...] = x_ref[...] * dc_ref[...]

    return pl.pallas_call(
        body,
        grid=(n // rb,),
        in_specs=[
            pl.BlockSpec((rb, d), lambda i: (i, 0)),
            pl.BlockSpec((rb, 1), lambda i: (i, 0)),
        ],
        out_specs=pl.BlockSpec((rb, d), lambda i: (i, 0)),
        out_shape=jax.ShapeDtypeStruct((n, d), jnp.float32),
    )(x, dinv_col)


def _agg_call(xin, src3, dst3, ew3, n_pad, d, nchunk):
    rpt = n_pad // NS

    stripe_chunks = []
    off = 0
    while off < rpt:
        sz = min(C, rpt - off)
        stripe_chunks.append((off, sz))
        off += sz

    M = 4

    def body(x_hbm, src_hbm, dst_hbm, ew_hbm, out_hbm,
             agg_sh, src_r, dst_r, ew_r, rows, gsem0, gsem1, msem):
        gsem = (gsem0, gsem1)
        cid = lax.axis_index("c")
        sid = lax.axis_index("s")
        wid = sid * NC + cid

        def meta_copies(g):
            slot = lax.rem(g, M)
            return (
                pltpu.make_async_copy(src_hbm.at[wid, g], src_r.at[slot],
                                      msem),
                pltpu.make_async_copy(dst_hbm.at[wid, g], dst_r.at[slot],
                                      msem),
                pltpu.make_async_copy(ew_hbm.at[wid, g], ew_r.at[slot],
                                      msem),
            )

        def issue_meta(g):
            for cp in meta_copies(g):
                cp.start()

        def wait_meta(g):
            for cp in meta_copies(g):
                cp.wait()

        def gather_copy(g, b):
            return pltpu.make_async_copy(
                x_hbm.at[src_r.at[lax.rem(g, M)]], rows.at[b], gsem[b])

        def zloop(r, carry):
            for l in range(d // LANES):
                rows[0, r, pl.ds(l * LANES, LANES)] = jnp.zeros(
                    (LANES,), jnp.float32)
            return carry
        lax.fori_loop(0, C, zloop, 0)
        for off, sz in stripe_chunks:
            pltpu.sync_copy(rows.at[0, pl.ds(0, sz)],
                            agg_sh.at[pl.ds(sid * rpt + off, sz)])
        issue_meta(0)
        issue_meta(1)
        issue_meta(2)
        wait_meta(0)
        gather_copy(0, 0).start()
        plsc.subcore_barrier()

        def outer(i, carry):
            for b in range(2):
                g = i * 2 + b
                slot = lax.rem(g, M)
                gather_copy(g, b).wait()

                @pl.when(g + 1 < nchunk)
                def _():
                    wait_meta(g + 1)
                    gather_copy(g + 1, 1 - b).start()

                def rloop(blk, c2):
                    nv = ew_r[slot, pl.ds(blk * LANES, LANES)]
                    for j in range(LANES):
                        r = blk * LANES + j
                        nrm = nv[j]
                        for l in range(d // LANES):
                            sl = pl.ds(l * LANES, LANES)
                            rows[b, r, sl] = rows[b, r, sl] * nrm
                    return c2
                lax.fori_loop(0, C // LANES, rloop, 0)
                pltpu.sync_copy(rows.at[b], agg_sh.at[dst_r.at[slot]],
                                add=True)

                @pl.when(g + 3 < nchunk)
                def _():
                    issue_meta(g + 3)
            return carry
        lax.fori_loop(0, nchunk // 2, outer, 0)
        plsc.subcore_barrier()
        for off, sz in stripe_chunks:
            pltpu.sync_copy(agg_sh.at[pl.ds(sid * rpt + off, sz)],
                            rows.at[0, pl.ds(0, sz)])
            pltpu.sync_copy(rows.at[0, pl.ds(0, sz)],
                            out_hbm.at[cid, pl.ds(sid * rpt + off, sz)])

    return pl.kernel(
        body,
        out_type=jax.ShapeDtypeStruct((NC, n_pad, d), jnp.float32),
        mesh=_sc_mesh(),
        compiler_params=pltpu.CompilerParams(needs_layout_passes=False),
        scratch_types=[
            pltpu.VMEM_SHARED((n_pad, d), jnp.float32),
            pltpu.VMEM((M, C), jnp.int32),
            pltpu.VMEM((M, C), jnp.int32),
            pltpu.VMEM((M, C), jnp.float32),
            pltpu.VMEM((2, C, d), jnp.float32),
        ] + [pltpu.SemaphoreType.DMA] * 3,
    )(xin, src3, dst3, ew3)


def _mm_call(p, xp, dinv_col, w, brow, relu_prescale, n, d):
    rb = 1000
    grid = (n // rb,)

    def body(p_ref, x_ref, dc_ref, w_ref, b_ref, o_ref):
        acc = (p_ref[0] + p_ref[1] + x_ref[...]) * dc_ref[...]
        y = jnp.dot(acc, w_ref[...], preferred_element_type=jnp.float32)
        y = y + b_ref[...]
        if relu_prescale:
            y = jnp.maximum(y, 0.0) * dc_ref[...]
        o_ref[...] = y

    return pl.pallas_call(
        body,
        grid=grid,
        in_specs=[
            pl.BlockSpec((NC, rb, d), lambda i: (0, i, 0)),
            pl.BlockSpec((rb, d), lambda i: (i, 0)),
            pl.BlockSpec((rb, 1), lambda i: (i, 0)),
            pl.BlockSpec((d, d), lambda i: (0, 0)),
            pl.BlockSpec((1, d), lambda i: (0, 0)),
        ],
        out_specs=pl.BlockSpec((rb, d), lambda i: (i, 0)),
        out_shape=jax.ShapeDtypeStruct((n, d), jnp.float32),
    )(p, xp, dinv_col, w, brow)


def kernel(x, edge_weight, W1, b1, W2, b2, edge_index):
    n, d = x.shape
    e = edge_index.shape[1]
    assert n % NS == 0 and d % LANES == 0
    nchunk = (e + NW * C - 1) // (NW * C)
    nchunk = ((nchunk + 1) // 2) * 2
    ept = nchunk * C
    e_pad = ept * NW
    n_pad = ((n + NS * LANES - 1) // (NS * LANES)) * (NS * LANES)
    n_agg = ((n + NS * 8 - 1) // (NS * 8)) * (NS * 8)

    src = edge_index[0]
    dst = edge_index[1]
    npad_e = e_pad - e
    pad_ids = (jnp.arange(npad_e, dtype=jnp.int32) % n)
    src3 = jnp.concatenate([src, pad_ids]).reshape(NW, nchunk, C)
    dst3 = jnp.concatenate([dst, pad_ids]).reshape(NW, nchunk, C)
    ew3 = jnp.concatenate(
        [edge_weight, jnp.zeros((npad_e,), jnp.float32)]).reshape(NW, nchunk, C)

    degp = _deg_call(dst3, ew3, n_pad, nchunk).reshape(NC, n_pad)
    dinv, dinv2 = _dinv_call(degp, n_pad)

    dinv_col = dinv[:n].reshape(n, 1)
    b1row = b1.reshape(1, d)
    b2row = b2.reshape(1, d)

    xp = _prescale_call(x, dinv_col, n, d)
    p1 = _agg_call(xp, src3, dst3, ew3, n_agg, d, nchunk)
    hp = _mm_call(p1, xp, dinv_col, W1, b1row, True, n, d)
    p2 = _agg_call(hp, src3, dst3, ew3, n_agg, d, nchunk)
    out = _mm_call(p2, hp, dinv_col, W2, b2row, False, n, d)
    return out
